# baseline (speedup 1.0000x reference)
import functools

import jax
import jax.numpy as jnp
from jax import lax
from jax.experimental import pallas as pl
from jax.experimental.pallas import tpu as pltpu
from jax.experimental.pallas import tpu_sc as plsc

S = 50
NB = 16384
D = 64
BT = 128
V = 1000000


def _mesh():
    return plsc.VectorSubcoreMesh(core_axis_name="c", subcore_axis_name="s")


@functools.cache
def _make_pack():
    info = plsc.get_sparse_core_info()
    NW = info.num_cores * info.num_subcores
    NC = info.num_cores
    CW = 128
    n_full = (V // CW)
    per_w = n_full // NW
    n_extra = n_full - per_w * NW

    @functools.partial(
        pl.kernel,
        mesh=_mesh(),
        compiler_params=pltpu.CompilerParams(
            use_tc_tiling_on_sc=True, needs_layout_passes=False
        ),
        out_type=jax.ShapeDtypeStruct((V // 2, 128), jnp.float32),
        scratch_types=[
            [pltpu.VMEM((D, CW), jnp.float32)] * 2,
            [pltpu.VMEM((CW // 2, 128), jnp.float32)] * 2,
            pltpu.VMEM((D, 64), jnp.float32),
            pltpu.VMEM((32, 128), jnp.float32),
            [pltpu.SemaphoreType.DMA] * 2,
            [pltpu.SemaphoreType.DMA] * 2,
        ],
    )
    def pack_kernel(emt, t128, svs, dvs, sv_t, dv_t, gsems, ssems):
        wid = lax.axis_index("s") * NC + lax.axis_index("c")
        iota = lax.iota(jnp.int32, 16)

        def load(u, p):
            u = jnp.minimum(u, n_full - 1)
            return pltpu.make_async_copy(
                emt.at[:, pl.ds(u * CW, CW)], svs[p], gsems[p]
            )

        def store(u, p):
            return pltpu.make_async_copy(
                dvs[p], t128.at[pl.ds(u * (CW // 2), CW // 2)], ssems[p]
            )

        def transpose(sv, dv, nv):
            def jloop(jv, carry2):
                pj = (iota + jv) & 15
                for d0 in range(0, D, 16):
                    dvec = d0 + pj
                    for v0 in range(0, nv, 16):
                        vv = v0 + iota
                        val = plsc.load_gather(sv, [dvec, vv])
                        plsc.store_scatter(
                            dv, [vv >> 1, ((vv & 1) << 6) + dvec], val
                        )
                return carry2

            lax.fori_loop(0, 16, jloop, 0)

        load(wid, 0).start()

        def step(i, carry):
            for b in range(2):
                j = 2 * i + b
                u = wid + NW * j
                load(jnp.minimum(u + NW, wid + NW * (per_w - 1)), 1 - b).start()
                load(u, b).wait()

                @pl.when(j >= 2)
                def _():
                    store(u - 2 * NW, b).wait()

                transpose(svs[b], dvs[b], CW)
                store(u, b).start()
            return carry

        lax.fori_loop(0, per_w // 2, step, 0)
        load(wid + NW * (per_w - 1), 0).wait()
        store(wid + NW * (per_w - 2), 0).wait()
        store(wid + NW * (per_w - 1), 1).wait()

        @pl.when(wid < n_extra)
        def _():
            u = per_w * NW + wid
            pltpu.sync_copy(emt.at[:, pl.ds(u * CW, CW)], svs[0])
            transpose(svs[0], dvs[0], CW)
            pltpu.sync_copy(dvs[0], t128.at[pl.ds(u * (CW // 2), CW // 2)])

        @pl.when(wid == n_extra)
        def _():
            pltpu.sync_copy(emt.at[:, pl.ds(n_full * CW, 64)], sv_t)
            transpose(sv_t, dv_t, 64)
            pltpu.sync_copy(dv_t, t128.at[pl.ds(n_full * (CW // 2), 32)])

    return pack_kernel


@functools.cache
def _make_gather():
    info = plsc.get_sparse_core_info()
    NC = info.num_cores
    NW = NC * info.num_subcores
    cpw = NB // NW
    kpw = cpw // BT
    n_units = S * kpw

    @functools.partial(
        pl.kernel,
        mesh=_mesh(),
        compiler_params=pltpu.CompilerParams(
            use_tc_tiling_on_sc=False, needs_layout_passes=False
        ),
        out_type=jax.ShapeDtypeStruct((S, D // 8, NB // BT, 8 * BT), jnp.float32),
        scratch_types=[
            pltpu.VMEM((S, cpw), jnp.int32),
            pltpu.VMEM((S, cpw), jnp.int32),
            [pltpu.VMEM((BT, 128), jnp.float32)] * 2,
            [pltpu.VMEM((D // 8, 8 * BT), jnp.float32)] * 2,
            [pltpu.SemaphoreType.DMA] * 2,
            [pltpu.SemaphoreType.DMA] * 2,
        ],
    )
    def gather_kernel(t128, idx2, out5, idx_v, idxg_v, rows, tiles, gsems, ssems):
        wid = lax.axis_index("s") * NC + lax.axis_index("c")
        col0 = wid * cpw

        pltpu.sync_copy(idx2.at[:, pl.ds(col0, cpw)], idx_v)

        def shift(j, carry):
            s = j // (cpw // 16)
            c = (j % (cpw // 16)) * 16
            idxg_v[s, pl.ds(c, 16)] = lax.shift_right_logical(
                idx_v[s, pl.ds(c, 16)], 1
            )
            return carry

        lax.fori_loop(0, S * (cpw // 16), shift, 0)

        iota = lax.iota(jnp.int32, 16)

        def unit_su(u):
            return u // kpw, u % kpw

        def gather(u, p):
            s, k = unit_su(u)
            return pltpu.make_async_copy(
                t128.at[idxg_v.at[s, pl.ds(k * BT, BT)]], rows[p], gsems[p]
            )

        def store(u, p):
            s, k = unit_su(u)
            return pltpu.make_async_copy(
                tiles[p], out5.at[s, :, wid * kpw + k], ssems[p]
            )

        gather(0, 0).start()

        def pair(i, carry):
            for b in range(2):
                u = i * 2 + b
                gather(jnp.minimum(u + 1, n_units - 1), 1 - b).start()
                gather(u, b).wait()

                @pl.when(u >= 2)
                def _():
                    store(u - 2, b).wait()

                s, k = unit_su(u)
                rb = rows[b]
                tb = tiles[b]

                def jloop(jv, carry2):
                    pj = (iota + jv) & 15
                    in0 = (pj & 7) * BT + iota
                    pj3 = pj >> 3
                    for t0 in range(0, BT, 16):
                        voff = (idx_v[s, pl.ds(k * BT + t0, 16)] & 1) << 6
                        rowv = t0 + iota
                        for d0 in range(0, D, 16):
                            colv = voff + (pj + d0)
                            v = plsc.load_gather(rb, [rowv, colv])
                            plsc.store_scatter(
                                tb, [pj3 + (d0 // 8), in0 + t0], v
                            )
                    return carry2

                lax.fori_loop(0, 16, jloop, 0)
                store(u, b).start()
            return carry

        lax.fori_loop(0, n_units // 2, pair, 0)

        gather(n_units - 1, 0).wait()
        store(n_units - 2, 0).wait()
        store(n_units - 1, 1).wait()

    return gather_kernel


def kernel(token_ids, embedding_matrix):
    t128 = _make_pack()(embedding_matrix.T)
    idx2 = token_ids.astype(jnp.int32).T
    out5 = _make_gather()(t128, idx2)
    out5 = out5.reshape(S, D // 8, NB // BT, 8, BT)
    return out5.transpose(2, 4, 0, 1, 3).reshape(NB, S, D)


# speedup vs baseline: 1.1610x; 1.1610x over previous
import functools

import jax
import jax.numpy as jnp
from jax import lax
from jax.experimental import pallas as pl
from jax.experimental.pallas import tpu as pltpu
from jax.experimental.pallas import tpu_sc as plsc

S = 50
NB = 16384
D = 64
BT = 128
V = 1000000


def _mesh():
    return plsc.VectorSubcoreMesh(core_axis_name="c", subcore_axis_name="s")


@functools.cache
def _make_pack():
    info = plsc.get_sparse_core_info()
    NW = info.num_cores * info.num_subcores
    NC = info.num_cores
    CW = 128
    n_full = (V // CW)
    per_w = n_full // NW
    n_extra = n_full - per_w * NW

    @functools.partial(
        pl.kernel,
        mesh=_mesh(),
        compiler_params=pltpu.CompilerParams(
            use_tc_tiling_on_sc=True, needs_layout_passes=False
        ),
        out_type=jax.ShapeDtypeStruct((V // 2, 128), jnp.float32),
        scratch_types=[
            [pltpu.VMEM((D, CW), jnp.float32)] * 2,
            [pltpu.VMEM((CW // 2, 128), jnp.float32)] * 2,
            pltpu.VMEM((D, 64), jnp.float32),
            pltpu.VMEM((32, 128), jnp.float32),
            [pltpu.SemaphoreType.DMA] * 2,
            [pltpu.SemaphoreType.DMA] * 2,
        ],
    )
    def pack_kernel(emt, t128, svs, dvs, sv_t, dv_t, gsems, ssems):
        wid = lax.axis_index("s") * NC + lax.axis_index("c")
        iota = lax.iota(jnp.int32, 16)

        def load(u, p):
            u = jnp.minimum(u, n_full - 1)
            return pltpu.make_async_copy(
                emt.at[:, pl.ds(u * CW, CW)], svs[p], gsems[p]
            )

        def store(u, p):
            return pltpu.make_async_copy(
                dvs[p], t128.at[pl.ds(u * (CW // 2), CW // 2)], ssems[p]
            )

        def transpose(sv, dv, nv):
            def jloop(jv, carry2):
                pj = (iota + jv) & 15
                for d0 in range(0, D, 16):
                    dvec = d0 + pj
                    for v0 in range(0, nv, 16):
                        vv = v0 + iota
                        val = plsc.load_gather(sv, [dvec, vv])
                        plsc.store_scatter(
                            dv, [vv >> 1, ((vv & 1) << 6) + dvec], val
                        )
                return carry2

            lax.fori_loop(0, 16, jloop, 0)

        load(wid, 0).start()

        def step(i, carry):
            for b in range(2):
                j = 2 * i + b
                u = wid + NW * j
                load(jnp.minimum(u + NW, wid + NW * (per_w - 1)), 1 - b).start()
                load(u, b).wait()

                @pl.when(j >= 2)
                def _():
                    store(u - 2 * NW, b).wait()

                transpose(svs[b], dvs[b], CW)
                store(u, b).start()
            return carry

        lax.fori_loop(0, per_w // 2, step, 0)
        load(wid + NW * (per_w - 1), 0).wait()
        store(wid + NW * (per_w - 2), 0).wait()
        store(wid + NW * (per_w - 1), 1).wait()

        @pl.when(wid < n_extra)
        def _():
            u = per_w * NW + wid
            pltpu.sync_copy(emt.at[:, pl.ds(u * CW, CW)], svs[0])
            transpose(svs[0], dvs[0], CW)
            pltpu.sync_copy(dvs[0], t128.at[pl.ds(u * (CW // 2), CW // 2)])

        @pl.when(wid == n_extra)
        def _():
            pltpu.sync_copy(emt.at[:, pl.ds(n_full * CW, 64)], sv_t)
            transpose(sv_t, dv_t, 64)
            pltpu.sync_copy(dv_t, t128.at[pl.ds(n_full * (CW // 2), 32)])

    return pack_kernel


@functools.cache
def _make_gather():
    info = plsc.get_sparse_core_info()
    NC = info.num_cores
    NW = NC * info.num_subcores
    cpw = NB // NW
    kpw = cpw // BT
    n_units = S * kpw

    @functools.partial(
        pl.kernel,
        mesh=_mesh(),
        compiler_params=pltpu.CompilerParams(
            use_tc_tiling_on_sc=False, needs_layout_passes=False
        ),
        out_type=jax.ShapeDtypeStruct((S, D // 8, NB // BT, 8 * BT), jnp.float32),
        scratch_types=[
            pltpu.VMEM((S, cpw), jnp.int32),
            [pltpu.VMEM((BT, D), jnp.float32)] * 2,
            [pltpu.VMEM((D // 8, 8 * BT), jnp.float32)] * 2,
            [pltpu.SemaphoreType.DMA] * 2,
            [pltpu.SemaphoreType.DMA] * 2,
        ],
    )
    def gather_kernel(t64, idx2, out5, idx_v, rows, tiles, gsems, ssems):
        wid = lax.axis_index("s") * NC + lax.axis_index("c")
        col0 = wid * cpw

        pltpu.sync_copy(idx2.at[:, pl.ds(col0, cpw)], idx_v)

        iota = lax.iota(jnp.int32, 16)

        def unit_su(u):
            return u // kpw, u % kpw

        def gather(u, p):
            s, k = unit_su(u)
            return pltpu.make_async_copy(
                t64.at[idx_v.at[s, pl.ds(k * BT, BT)]], rows[p], gsems[p]
            )

        def store(u, p):
            s, k = unit_su(u)
            return pltpu.make_async_copy(
                tiles[p], out5.at[s, :, wid * kpw + k], ssems[p]
            )

        gather(0, 0).start()

        def pair(i, carry):
            for b in range(2):
                u = i * 2 + b
                gather(jnp.minimum(u + 1, n_units - 1), 1 - b).start()
                gather(u, b).wait()

                @pl.when(u >= 2)
                def _():
                    store(u - 2, b).wait()

                rb = rows[b]
                tb = tiles[b]

                def jloop(jv, carry2):
                    pj = (iota + jv) & 15
                    in0 = (pj & 7) * BT + iota
                    pj3 = pj >> 3
                    for d0 in range(0, D, 16):
                        cold = pj + d0
                        i0d = pj3 + (d0 // 8)
                        for t0 in range(0, BT, 16):
                            v = plsc.load_gather(rb, [t0 + iota, cold])
                            plsc.store_scatter(tb, [i0d, in0 + t0], v)
                    return carry2

                lax.fori_loop(0, 16, jloop, 0)
                store(u, b).start()
            return carry

        lax.fori_loop(0, n_units // 2, pair, 0)

        gather(n_units - 1, 0).wait()
        store(n_units - 2, 0).wait()
        store(n_units - 1, 1).wait()

    return gather_kernel


def kernel(token_ids, embedding_matrix):
    t128 = _make_pack()(embedding_matrix.T)
    idx2 = token_ids.astype(jnp.int32).T
    out5 = _make_gather()(t128.reshape(V, D), idx2)
    out5 = out5.reshape(S, D // 8, NB // BT, 8, BT)
    return out5.transpose(2, 4, 0, 1, 3).reshape(NB, S, D)
